# baseline (speedup 1.0000x reference)
import functools

import jax
import jax.numpy as jnp
from jax import lax
from jax.experimental import pallas as pl
from jax.experimental.pallas import tpu as pltpu
from jax.experimental.pallas import tpu_sc as plsc

NC = 2
NS = 16
K = 128
GROUP = 8


def _pads(n_nodes, n_edges):
    tiles = NC * NS
    n_chunks = pl.cdiv(pl.cdiv(n_edges, tiles), K)
    ept = n_chunks * K
    e_pad = ept * tiles
    tile_n = pl.cdiv(n_nodes + 1, NS * 8) * 8
    n_pad = tile_n * NS
    return ept, e_pad, tile_n, n_pad


def _mesh():
    return plsc.VectorSubcoreMesh(
        core_axis_name="c", subcore_axis_name="s", num_cores=NC, num_subcores=NS
    )


def _make_deg_kernel(e_pad, tile_n, n_pad, ept):
    n_chunks = ept // K
    zlen = pl.cdiv(tile_n, 16) * 16

    @functools.partial(
        pl.kernel,
        out_type=jax.ShapeDtypeStruct((NC * n_pad,), jnp.float32),
        mesh=_mesh(),
        scratch_types=[
            pltpu.VMEM((K,), jnp.float32),
            pltpu.VMEM((zlen,), jnp.float32),
            pltpu.VMEM((K,), jnp.int32),
            pltpu.VMEM_SHARED((n_pad,), jnp.float32),
        ],
    )
    def deg_kernel(dst_hbm, out_hbm, ones_v, zeros_v, didx_v, deg_sh):
        c = lax.axis_index("c")
        s = lax.axis_index("s")
        wid = c * NS + s
        for i in range(K // 16):
            ones_v[pl.ds(i * 16, 16)] = jnp.ones((16,), jnp.float32)
        for i in range(zlen // 16):
            zeros_v[pl.ds(i * 16, 16)] = jnp.zeros((16,), jnp.float32)
        pltpu.sync_copy(zeros_v.at[pl.ds(0, tile_n)],
                        deg_sh.at[pl.ds(s * tile_n, tile_n)])
        plsc.subcore_barrier()

        def body(j, carry):
            base = wid * ept + j * K
            pltpu.sync_copy(dst_hbm.at[pl.ds(base, K)], didx_v)
            pltpu.sync_copy(ones_v, deg_sh.at[didx_v], add=True)
            return carry

        lax.fori_loop(0, n_chunks, body, 0)
        plsc.subcore_barrier()
        pltpu.sync_copy(deg_sh.at[pl.ds(s * tile_n, tile_n)],
                        zeros_v.at[pl.ds(0, tile_n)])
        pltpu.sync_copy(zeros_v.at[pl.ds(0, tile_n)],
                        out_hbm.at[pl.ds(c * n_pad + s * tile_n, tile_n)])

    return deg_kernel


def _make_scatter_kernel(e_pad, tile_n, n_pad, ept, d):
    n_chunks = ept // K
    zfull, zrem = tile_n // K, tile_n % K

    @functools.partial(
        pl.kernel,
        out_type=jax.ShapeDtypeStruct((NC, n_pad, d), jnp.float32),
        mesh=_mesh(),
        scratch_types=[
            pltpu.VMEM((K,), jnp.int32),
            pltpu.VMEM((K,), jnp.int32),
            pltpu.VMEM((K, d), jnp.float32),
            pltpu.VMEM((K, d), jnp.float32),
            pltpu.VMEM_SHARED((n_pad, d), jnp.float32),
            pltpu.SemaphoreType.DMA,
            pltpu.SemaphoreType.DMA,
        ],
    )
    def scatter_kernel(table_hbm, src_hbm, dst_hbm, out_hbm,
                       sidx_v, didx_v, rows_v, rows2_v, acc_sh, sem, sem2):
        c = lax.axis_index("c")
        s = lax.axis_index("s")
        wid = c * NS + s

        def zrow(i, carry):
            for q in range(d // 16):
                rows_v[i, pl.ds(q * 16, 16)] = jnp.zeros((16,), jnp.float32)
            return carry

        lax.fori_loop(0, K, zrow, 0)
        zdescs = []
        for q in range(zfull):
            zdescs.append(pltpu.async_copy(
                rows_v, acc_sh.at[pl.ds(s * tile_n + q * K, K)], sem))
        if zrem:
            zdescs.append(pltpu.async_copy(
                rows_v.at[pl.ds(0, zrem)],
                acc_sh.at[pl.ds(s * tile_n + zfull * K, zrem)], sem))
        for zd in zdescs:
            zd.wait()
        plsc.subcore_barrier()

        def body(j, carry):
            base = wid * ept + j * K
            pltpu.sync_copy(src_hbm.at[pl.ds(base, K)], sidx_v)
            pltpu.sync_copy(dst_hbm.at[pl.ds(base, K)], didx_v)
            pltpu.async_copy(table_hbm.at[sidx_v], rows_v, sem).wait()
            pltpu.sync_copy(rows_v, acc_sh.at[didx_v], add=True)
            return carry

        lax.fori_loop(0, n_chunks, body, 0)
        plsc.subcore_barrier()
        spans = [(q * K, K) for q in range(zfull)]
        if zrem:
            spans.append((zfull * K, zrem))
        bufs = (rows_v, rows2_v)
        wdesc = [None, None]
        for i, (off, ln) in enumerate(spans):
            b = i % 2
            if wdesc[b] is not None:
                wdesc[b].wait()
            pltpu.async_copy(acc_sh.at[pl.ds(s * tile_n + off, ln)],
                             bufs[b].at[pl.ds(0, ln)], sem).wait()
            wdesc[b] = pltpu.async_copy(
                bufs[b].at[pl.ds(0, ln)],
                out_hbm.at[c, pl.ds(s * tile_n + off, ln)], sem2)
        for wd in wdesc:
            if wd is not None:
                wd.wait()

    return scatter_kernel


def _deg_vecs(degp_ref, n):
    deg = 1.0 + degp_ref[0, :n] + degp_ref[1, :n]
    dinv = lax.rsqrt(deg)
    return deg, dinv


def _prep_body(x_ref, w_ref, b_ref, degp_ref, hs_ref, st_ref):
    n = x_ref.shape[0]
    deg, dinv = _deg_vecs(degp_ref, n)
    h = jnp.dot(x_ref[...], w_ref[...], preferred_element_type=jnp.float32)
    hs_ref[pl.ds(0, n), :] = h * dinv[:, None]
    st_ref[...] = h * (1.0 / deg)[:, None] + b_ref[...][None, :]


def _mid_body(acc_ref, st1_ref, degp_ref, w_ref, b_ref, hs_ref, st_ref):
    n = st1_ref.shape[0]
    deg, dinv = _deg_vecs(degp_ref, n)
    pre = dinv[:, None] * (acc_ref[0, :n] + acc_ref[1, :n]) + st1_ref[...]
    h1 = jnp.maximum(pre, 0.0)
    h = jnp.dot(h1, w_ref[...], preferred_element_type=jnp.float32)
    hs_ref[pl.ds(0, n), :] = h * dinv[:, None]
    st_ref[...] = h * (1.0 / deg)[:, None] + b_ref[...][None, :]


def _final_body(acc_ref, st2_ref, degp_ref, z_ref):
    n = z_ref.shape[0]
    _, dinv = _deg_vecs(degp_ref, n)
    z_ref[...] = dinv[:, None] * (acc_ref[0, :n] + acc_ref[1, :n]) + st2_ref[...]


def kernel(x, edge_index, W1, b1, W2, b2):
    n, d = x.shape
    e = edge_index.shape[1]
    ept, e_pad, tile_n, n_pad = _pads(n, e)

    src = edge_index[0].astype(jnp.int32)
    dst = edge_index[1].astype(jnp.int32)
    src_p = jnp.concatenate([src, jnp.zeros((e_pad - e,), jnp.int32)])
    dst_p = jnp.concatenate(
        [dst, jnp.full((e_pad - e,), n_pad - 1, jnp.int32)])

    degp = _make_deg_kernel(e_pad, tile_n, n_pad, ept)(dst_p)
    degp = degp.reshape(NC, n_pad)
    scatter = _make_scatter_kernel(e_pad, tile_n, n_pad, ept, d)

    f32 = jnp.float32
    nd_pad = jax.ShapeDtypeStruct((n_pad, d), f32)
    nd = jax.ShapeDtypeStruct((n, d), f32)
    hs1, st1 = pl.pallas_call(
        _prep_body, out_shape=[nd_pad, nd])(x, W1, b1, degp)
    acc1 = scatter(hs1, src_p, dst_p)
    hs2, st2 = pl.pallas_call(
        _mid_body, out_shape=[nd_pad, nd])(acc1, st1, degp, W2, b2)
    acc2 = scatter(hs2, src_p, dst_p)
    return pl.pallas_call(
        _final_body, out_shape=nd)(acc2, st2, degp)


# speedup vs baseline: 1.0064x; 1.0064x over previous
import functools

import jax
import jax.numpy as jnp
from jax import lax
from jax.experimental import pallas as pl
from jax.experimental.pallas import tpu as pltpu
from jax.experimental.pallas import tpu_sc as plsc

NC = 2
NS = 16
K = 128


def _pads(n_nodes, n_edges):
    tiles = NC * NS
    n_chunks = pl.cdiv(pl.cdiv(n_edges, tiles), K)
    ept = n_chunks * K
    e_pad = ept * tiles
    tile_n = pl.cdiv(n_nodes + 1, NS * 8) * 8
    n_pad = tile_n * NS
    return ept, e_pad, tile_n, n_pad


def _mesh():
    return plsc.VectorSubcoreMesh(
        core_axis_name="c", subcore_axis_name="s", num_cores=NC, num_subcores=NS
    )


def _make_deg_kernel(e_pad, tile_n, n_pad, ept):
    n_chunks = ept // K
    zlen = pl.cdiv(tile_n, 16) * 16

    @functools.partial(
        pl.kernel,
        out_type=jax.ShapeDtypeStruct((NC * n_pad,), jnp.float32),
        mesh=_mesh(),
        scratch_types=[
            pltpu.VMEM((K,), jnp.float32),
            pltpu.VMEM((zlen,), jnp.float32),
            pltpu.VMEM((K,), jnp.int32),
            pltpu.VMEM_SHARED((n_pad,), jnp.float32),
        ],
    )
    def deg_kernel(dst_hbm, out_hbm, ones_v, zeros_v, didx_v, deg_sh):
        c = lax.axis_index("c")
        s = lax.axis_index("s")
        wid = c * NS + s
        for i in range(K // 16):
            ones_v[pl.ds(i * 16, 16)] = jnp.ones((16,), jnp.float32)
        for i in range(zlen // 16):
            zeros_v[pl.ds(i * 16, 16)] = jnp.zeros((16,), jnp.float32)
        pltpu.sync_copy(zeros_v.at[pl.ds(0, tile_n)],
                        deg_sh.at[pl.ds(s * tile_n, tile_n)])
        plsc.subcore_barrier()

        def body(j, carry):
            base = wid * ept + j * K
            pltpu.sync_copy(dst_hbm.at[pl.ds(base, K)], didx_v)
            pltpu.sync_copy(ones_v, deg_sh.at[didx_v], add=True)
            return carry

        lax.fori_loop(0, n_chunks, body, 0)
        plsc.subcore_barrier()
        pltpu.sync_copy(deg_sh.at[pl.ds(s * tile_n, tile_n)],
                        zeros_v.at[pl.ds(0, tile_n)])
        pltpu.sync_copy(zeros_v.at[pl.ds(0, tile_n)],
                        out_hbm.at[pl.ds(c * n_pad + s * tile_n, tile_n)])

    return deg_kernel


def _make_scatter_kernel(e_pad, tile_n, n_pad, ept, d):
    n_chunks = ept // K
    zfull, zrem = tile_n // K, tile_n % K

    @functools.partial(
        pl.kernel,
        out_type=jax.ShapeDtypeStruct((NC, n_pad, d), jnp.float32),
        mesh=_mesh(),
        scratch_types=[
            pltpu.VMEM((K,), jnp.int32),
            pltpu.VMEM((K,), jnp.int32),
            pltpu.VMEM((K, d), jnp.float32),
            pltpu.VMEM((K, d), jnp.float32),
            pltpu.VMEM_SHARED((n_pad, d), jnp.float32),
            pltpu.SemaphoreType.DMA,
            pltpu.SemaphoreType.DMA,
        ],
    )
    def scatter_kernel(table_hbm, src_hbm, dst_hbm, out_hbm,
                       sidx_v, didx_v, rows_v, rows2_v, acc_sh, sem, sem2):
        c = lax.axis_index("c")
        s = lax.axis_index("s")
        wid = c * NS + s

        def zrow(i, carry):
            for q in range(d // 16):
                rows_v[i, pl.ds(q * 16, 16)] = jnp.zeros((16,), jnp.float32)
            return carry

        lax.fori_loop(0, K, zrow, 0)
        zdescs = []
        for q in range(zfull):
            zdescs.append(pltpu.async_copy(
                rows_v, acc_sh.at[pl.ds(s * tile_n + q * K, K)], sem))
        if zrem:
            zdescs.append(pltpu.async_copy(
                rows_v.at[pl.ds(0, zrem)],
                acc_sh.at[pl.ds(s * tile_n + zfull * K, zrem)], sem))
        for zd in zdescs:
            zd.wait()
        plsc.subcore_barrier()

        def body(j, carry):
            base = wid * ept + j * K
            pltpu.sync_copy(src_hbm.at[pl.ds(base, K)], sidx_v)
            pltpu.sync_copy(dst_hbm.at[pl.ds(base, K)], didx_v)
            pltpu.async_copy(table_hbm.at[sidx_v], rows_v, sem).wait()
            pltpu.sync_copy(rows_v, acc_sh.at[didx_v], add=True)
            return carry

        lax.fori_loop(0, n_chunks, body, 0)
        plsc.subcore_barrier()
        spans = [(q * K, K) for q in range(zfull)]
        if zrem:
            spans.append((zfull * K, zrem))
        bufs = (rows_v, rows2_v)
        wdesc = [None, None]
        for i, (off, ln) in enumerate(spans):
            b = i % 2
            if wdesc[b] is not None:
                wdesc[b].wait()
            pltpu.async_copy(acc_sh.at[pl.ds(s * tile_n + off, ln)],
                             bufs[b].at[pl.ds(0, ln)], sem).wait()
            wdesc[b] = pltpu.async_copy(
                bufs[b].at[pl.ds(0, ln)],
                out_hbm.at[c, pl.ds(s * tile_n + off, ln)], sem2)
        for wd in wdesc:
            if wd is not None:
                wd.wait()

    return scatter_kernel


def _deg_vecs(degp_ref):
    deg = 1.0 + degp_ref[0] + degp_ref[1]
    dinv = lax.rsqrt(deg)
    return deg, dinv


def _prep_body(x_ref, w_ref, b_ref, degp_ref, hs_ref, st_ref):
    deg, dinv = _deg_vecs(degp_ref)
    h = jnp.dot(x_ref[...], w_ref[...], preferred_element_type=jnp.float32)
    hs_ref[...] = h * dinv[:, None]
    st_ref[...] = h * (1.0 / deg)[:, None] + b_ref[...][None, :]


def _mid_body(acc_ref, st1_ref, degp_ref, w_ref, b_ref, hs_ref, st_ref):
    deg, dinv = _deg_vecs(degp_ref)
    pre = dinv[:, None] * (acc_ref[0] + acc_ref[1]) + st1_ref[...]
    h1 = jnp.maximum(pre, 0.0)
    h = jnp.dot(h1, w_ref[...], preferred_element_type=jnp.float32)
    hs_ref[...] = h * dinv[:, None]
    st_ref[...] = h * (1.0 / deg)[:, None] + b_ref[...][None, :]


def _final_body(acc_ref, st2_ref, degp_ref, z_ref):
    _, dinv = _deg_vecs(degp_ref)
    z_ref[...] = dinv[:, None] * (acc_ref[0] + acc_ref[1]) + st2_ref[...]


def kernel(x, edge_index, W1, b1, W2, b2):
    n, d = x.shape
    e = edge_index.shape[1]
    ept, e_pad, tile_n, n_pad = _pads(n, e)

    src = edge_index[0].astype(jnp.int32)
    dst = edge_index[1].astype(jnp.int32)
    src_p = jnp.concatenate([src, jnp.zeros((e_pad - e,), jnp.int32)])
    dst_p = jnp.concatenate(
        [dst, jnp.full((e_pad - e,), n_pad - 1, jnp.int32)])
    x_p = jnp.pad(x, ((0, n_pad - n), (0, 0)))

    degp = _make_deg_kernel(e_pad, tile_n, n_pad, ept)(dst_p)
    degp = degp.reshape(NC, n_pad)
    scatter = _make_scatter_kernel(e_pad, tile_n, n_pad, ept, d)

    f32 = jnp.float32
    nd = jax.ShapeDtypeStruct((n_pad, d), f32)
    hs1, st1 = pl.pallas_call(
        _prep_body, out_shape=[nd, nd])(x_p, W1, b1, degp)
    acc1 = scatter(hs1, src_p, dst_p)
    hs2, st2 = pl.pallas_call(
        _mid_body, out_shape=[nd, nd])(acc1, st1, degp, W2, b2)
    acc2 = scatter(hs2, src_p, dst_p)
    z_p = pl.pallas_call(
        _final_body, out_shape=nd)(acc2, st2, degp)
    return z_p[:n]
